# baseline (speedup 1.0000x reference)
import functools

import jax
import jax.numpy as jnp
from jax import lax
from jax.experimental import pallas as pl
from jax.experimental.pallas import tpu as pltpu
from jax.experimental.pallas import tpu_sc as plsc

N = 10000
NPAD = 10240
E = 320000
NF = 16
EF = 3
H = 128
OUT = 2
STEPS = 10

NC = 2
NS = 16
NW = NC * NS
E_PAD = 327680
EPW = E_PAD // NW
CH = 80
NCH = EPW // CH
RPT = NPAD // NS

TN = 1024
TE = 2560

_mesh = plsc.VectorSubcoreMesh(core_axis_name="c", subcore_axis_name="s")


@functools.partial(
    pl.kernel,
    mesh=_mesh,
    out_type=jax.ShapeDtypeStruct((NC, NPAD, H), jnp.float32),
    scratch_types=[
        pltpu.VMEM((NCH, CH), jnp.int32),
        pltpu.VMEM((CH, H), jnp.float32),
        pltpu.VMEM_SHARED((NPAD, H), jnp.float32),
    ],
)
def _sc_counts(dst2, ones_rows, zeros128, out, idx_v, ones_v, acc):
    cid = lax.axis_index("c")
    sid = lax.axis_index("s")
    wid = sid * NC + cid
    pltpu.sync_copy(zeros128, ones_v)
    for k in range(RPT // CH):
        pltpu.sync_copy(ones_v, acc.at[pl.ds(sid * RPT + k * CH, CH), :])
    pltpu.sync_copy(ones_rows, ones_v)
    pltpu.sync_copy(dst2.at[wid], idx_v)
    plsc.subcore_barrier()

    def body(j, carry):
        pltpu.sync_copy(ones_v, acc.at[idx_v.at[j]], add=True)
        return carry

    lax.fori_loop(0, NCH, body, 0)
    plsc.subcore_barrier()
    rows = pl.ds(sid * RPT, RPT)
    pltpu.sync_copy(acc.at[rows, :], out.at[cid, rows, :])


@functools.partial(
    pl.kernel,
    mesh=_mesh,
    out_type=jax.ShapeDtypeStruct((E_PAD, H), jnp.float32),
    scratch_types=[
        pltpu.VMEM((NCH, CH), jnp.int32),
        pltpu.VMEM((CH, H), jnp.float32),
        pltpu.SemaphoreType.DMA,
    ],
)
def _sc_gather(table, src2, out, idx_v, rows_v, sem):
    cid = lax.axis_index("c")
    sid = lax.axis_index("s")
    wid = sid * NC + cid
    pltpu.sync_copy(src2.at[wid], idx_v)

    def body(j, carry):
        base = wid * EPW + j * CH
        pltpu.async_copy(table.at[idx_v.at[j]], rows_v, sem).wait()
        pltpu.sync_copy(rows_v, out.at[pl.ds(base, CH), :])
        return carry

    lax.fori_loop(0, NCH, body, 0)


@functools.partial(
    pl.kernel,
    mesh=_mesh,
    out_type=jax.ShapeDtypeStruct((NC, NPAD, H), jnp.float32),
    scratch_types=[
        pltpu.VMEM((NCH, CH), jnp.int32),
        pltpu.VMEM((CH, H), jnp.float32),
        pltpu.VMEM_SHARED((NPAD, H), jnp.float32),
    ],
)
def _sc_scatter(msg, dst2, zeros128, out, idx_v, rows_v, acc):
    cid = lax.axis_index("c")
    sid = lax.axis_index("s")
    wid = sid * NC + cid
    pltpu.sync_copy(zeros128, rows_v)
    for k in range(RPT // CH):
        pltpu.sync_copy(rows_v, acc.at[pl.ds(sid * RPT + k * CH, CH), :])
    pltpu.sync_copy(dst2.at[wid], idx_v)
    plsc.subcore_barrier()

    def body(j, carry):
        base = wid * EPW + j * CH
        pltpu.sync_copy(msg.at[pl.ds(base, CH), :], rows_v)
        pltpu.sync_copy(rows_v, acc.at[idx_v.at[j]], add=True)
        return carry

    lax.fori_loop(0, NCH, body, 0)
    plsc.subcore_barrier()
    rows_o = pl.ds(sid * RPT, RPT)
    pltpu.sync_copy(acc.at[rows_o, :], out.at[cid, rows_o, :])


def _dot(a, b):
    return jnp.dot(a, b, preferred_element_type=jnp.float32)


def _full(shape):
    return pl.BlockSpec(shape, lambda i: tuple(0 for _ in shape))


def _edge_enc_body(ea, w1, b1, w2, b2, w3, b3, out):
    h = jnp.maximum(_dot(ea[...], w1[...]) + b1[...], 0.0)
    h = jnp.maximum(_dot(h, w2[...]) + b2[...], 0.0)
    out[...] = _dot(h, w3[...]) + b3[...]


def _tc_edge_enc(ea8, w1, b1, w2, b2, w3, b3):
    return pl.pallas_call(
        _edge_enc_body,
        grid=(E_PAD // TE,),
        in_specs=[pl.BlockSpec((TE, 8), lambda i: (i, 0)),
                  _full((8, H)), _full((1, H)),
                  _full((H, H)), _full((1, H)),
                  _full((H, H)), _full((1, H))],
        out_specs=pl.BlockSpec((TE, H), lambda i: (i, 0)),
        out_shape=jax.ShapeDtypeStruct((E_PAD, H), jnp.float32),
    )(ea8, w1, b1, w2, b2, w3, b3)


def _node_enc_body(x, c0, c1, w1, b1, w2, b2, w3, b3, wa, ba, emb, den, proj):
    col = lax.broadcasted_iota(jnp.int32, (TN, NF), 1)
    xm = jnp.where(col < 2, 0.0, x[...])
    h = jnp.maximum(_dot(xm, w1[...]) + b1[...], 0.0)
    h = jnp.maximum(_dot(h, w2[...]) + b2[...], 0.0)
    e = _dot(h, w3[...]) + b3[...]
    emb[...] = e
    cnt = c0[...] + c1[...]
    den[...] = jnp.broadcast_to(jnp.maximum(cnt[:, 0:1], 1.0), (TN, H))
    proj[...] = _dot(e, wa[...]) + ba[...]


def _tc_node_enc(x_p, c0, c1, w1, b1, w2, b2, w3, b3, wa, ba):
    return pl.pallas_call(
        _node_enc_body,
        grid=(NPAD // TN,),
        in_specs=[pl.BlockSpec((TN, NF), lambda i: (i, 0)),
                  pl.BlockSpec((TN, H), lambda i: (i, 0)),
                  pl.BlockSpec((TN, H), lambda i: (i, 0)),
                  _full((NF, H)), _full((1, H)),
                  _full((H, H)), _full((1, H)),
                  _full((H, H)), _full((1, H)),
                  _full((H, H)), _full((1, H))],
        out_specs=[pl.BlockSpec((TN, H), lambda i: (i, 0)),
                   pl.BlockSpec((TN, H), lambda i: (i, 0)),
                   pl.BlockSpec((TN, H), lambda i: (i, 0))],
        out_shape=[jax.ShapeDtypeStruct((NPAD, H), jnp.float32),
                   jax.ShapeDtypeStruct((NPAD, H), jnp.float32),
                   jax.ShapeDtypeStruct((NPAD, H), jnp.float32)],
    )(x_p, c0, c1, w1, b1, w2, b2, w3, b3, wa, ba)


def _msg_body(xjp, ee, wb, w2, b2, w3, b3, out):
    h1 = jnp.maximum(xjp[...] + _dot(ee[...], wb[...]), 0.0)
    h2 = jnp.maximum(_dot(h1, w2[...]) + b2[...], 0.0)
    out[...] = _dot(h2, w3[...]) + b3[...]


def _tc_msg(xjp, ee, wb, w2, b2, w3, b3):
    return pl.pallas_call(
        _msg_body,
        grid=(E_PAD // TE,),
        in_specs=[pl.BlockSpec((TE, H), lambda i: (i, 0)),
                  pl.BlockSpec((TE, H), lambda i: (i, 0)),
                  _full((H, H)), _full((H, H)), _full((1, H)),
                  _full((H, H)), _full((1, H))],
        out_specs=pl.BlockSpec((TE, H), lambda i: (i, 0)),
        out_shape=jax.ShapeDtypeStruct((E_PAD, H), jnp.float32),
    )(xjp, ee, wb, w2, b2, w3, b3)


def _update_body(emb, p0, p1, den, wa, ba, nemb, proj):
    e = emb[...] + (p0[...] + p1[...]) / den[...]
    nemb[...] = e
    proj[...] = _dot(e, wa[...]) + ba[...]


def _tc_update(emb, p0, p1, den, wa, ba):
    return pl.pallas_call(
        _update_body,
        grid=(NPAD // TN,),
        in_specs=[pl.BlockSpec((TN, H), lambda i: (i, 0)),
                  pl.BlockSpec((TN, H), lambda i: (i, 0)),
                  pl.BlockSpec((TN, H), lambda i: (i, 0)),
                  pl.BlockSpec((TN, H), lambda i: (i, 0)),
                  _full((H, H)), _full((1, H))],
        out_specs=[pl.BlockSpec((TN, H), lambda i: (i, 0)),
                   pl.BlockSpec((TN, H), lambda i: (i, 0))],
        out_shape=[jax.ShapeDtypeStruct((NPAD, H), jnp.float32),
                   jax.ShapeDtypeStruct((NPAD, H), jnp.float32)],
    )(emb, p0, p1, den, wa, ba)


def _final_body(emb, p0, p1, den, w1, b1, w2, b2, w3, b3, out):
    e = emb[...] + (p0[...] + p1[...]) / den[...]
    h = jnp.maximum(_dot(e, w1[...]) + b1[...], 0.0)
    h = jnp.maximum(_dot(h, w2[...]) + b2[...], 0.0)
    out[...] = _dot(h, w3[...]) + b3[...]


def _tc_final(emb, p0, p1, den, w1, b1, w2, b2, w3, b3):
    return pl.pallas_call(
        _final_body,
        grid=(NPAD // TN,),
        in_specs=[pl.BlockSpec((TN, H), lambda i: (i, 0)),
                  pl.BlockSpec((TN, H), lambda i: (i, 0)),
                  pl.BlockSpec((TN, H), lambda i: (i, 0)),
                  pl.BlockSpec((TN, H), lambda i: (i, 0)),
                  _full((H, H)), _full((1, H)),
                  _full((H, H)), _full((1, H)),
                  _full((H, H)), _full((1, H))],
        out_specs=pl.BlockSpec((TN, H), lambda i: (i, 0)),
        out_shape=jax.ShapeDtypeStruct((NPAD, H), jnp.float32),
    )(emb, p0, p1, den, w1, b1, w2, b2, w3, b3)


def _row(b):
    return b.reshape(1, H)


def kernel(x, edge_index, edge_attr, params):
    src = edge_index[0].astype(jnp.int32)
    dst = edge_index[1].astype(jnp.int32)
    src2 = jnp.pad(src, (0, E_PAD - E),
                   constant_values=NPAD - 1).reshape(NW, NCH, CH)
    dst2 = jnp.pad(dst, (0, E_PAD - E),
                   constant_values=NPAD - 1).reshape(NW, NCH, CH)
    x_p = jnp.pad(x, ((0, NPAD - N), (0, 0)))
    ea8 = jnp.pad(edge_attr, ((0, E_PAD - E), (0, 8 - EF)))

    nps = params["node"]
    eps = params["edge"]
    dec = params["dec"]
    procs = params["proc"]

    ew1 = jnp.pad(eps[0][0], ((0, 8 - EF), (0, 0)))
    dw3 = jnp.pad(dec[2][0], ((0, 0), (0, H - OUT)))
    db3 = jnp.pad(dec[2][1], ((0, H - OUT),))

    was = [p[0][0][:H] for p in procs]
    wbs = [p[0][0][H:] for p in procs]

    ones_rows = jnp.zeros((CH, H), jnp.float32).at[:, 0].set(1.0)
    zeros128 = jnp.zeros((CH, H), jnp.float32)

    cpart = _sc_counts(dst2, ones_rows, zeros128)
    c0, c1 = cpart[0], cpart[1]
    edge_emb = _tc_edge_enc(ea8, ew1, _row(eps[0][1]),
                            eps[1][0], _row(eps[1][1]),
                            eps[2][0], _row(eps[2][1]))
    emb, den, proj = _tc_node_enc(
        x_p, c0, c1,
        nps[0][0], _row(nps[0][1]),
        nps[1][0], _row(nps[1][1]),
        nps[2][0], _row(nps[2][1]),
        was[0], _row(procs[0][0][1]))

    out = None
    for s in range(STEPS):
        ps = procs[s]
        xjp = _sc_gather(proj, src2)
        msg = _tc_msg(xjp, edge_emb, wbs[s],
                      ps[1][0], _row(ps[1][1]),
                      ps[2][0], _row(ps[2][1]))
        part = _sc_scatter(msg, dst2, zeros128)
        p0, p1 = part[0], part[1]
        if s + 1 < STEPS:
            emb, proj = _tc_update(emb, p0, p1, den,
                                   was[s + 1], _row(procs[s + 1][0][1]))
        else:
            out = _tc_final(emb, p0, p1, den,
                            dec[0][0], _row(dec[0][1]),
                            dec[1][0], _row(dec[1][1]),
                            dw3, _row(db3))
    return out[:N, :OUT]


# speedup vs baseline: 1.5508x; 1.5508x over previous
import functools

import jax
import jax.numpy as jnp
from jax import lax
from jax.experimental import pallas as pl
from jax.experimental.pallas import tpu as pltpu
from jax.experimental.pallas import tpu_sc as plsc

N = 10000
NPAD = 10240
E = 320000
NF = 16
EF = 3
H = 128
OUT = 2
STEPS = 10

NC = 2
NS = 16
NW = NC * NS
E_PAD = 327680
EPW = E_PAD // NW
CH = 80
NCH = EPW // CH
RPT = NPAD // NS

TN = 1024
TE = 2560

_mesh = plsc.VectorSubcoreMesh(core_axis_name="c", subcore_axis_name="s")


@functools.partial(
    pl.kernel,
    mesh=_mesh,
    out_type=jax.ShapeDtypeStruct((NC, NPAD, H), jnp.float32),
    scratch_types=[
        pltpu.VMEM((NCH, CH), jnp.int32),
        pltpu.VMEM((CH, H), jnp.float32),
        pltpu.VMEM_SHARED((NPAD, H), jnp.float32),
    ],
)
def _sc_counts(dst2, ones_rows, zeros128, out, idx_v, ones_v, acc):
    cid = lax.axis_index("c")
    sid = lax.axis_index("s")
    wid = sid * NC + cid
    pltpu.sync_copy(zeros128, ones_v)
    for k in range(RPT // CH):
        pltpu.sync_copy(ones_v, acc.at[pl.ds(sid * RPT + k * CH, CH), :])
    pltpu.sync_copy(ones_rows, ones_v)
    pltpu.sync_copy(dst2.at[wid], idx_v)
    plsc.subcore_barrier()

    def body(j, carry):
        pltpu.sync_copy(ones_v, acc.at[idx_v.at[j]], add=True)
        return carry

    lax.fori_loop(0, NCH, body, 0)
    plsc.subcore_barrier()
    rows = pl.ds(sid * RPT, RPT)
    pltpu.sync_copy(acc.at[rows, :], out.at[cid, rows, :])


@functools.partial(
    pl.kernel,
    mesh=_mesh,
    out_type=jax.ShapeDtypeStruct((E_PAD, H), jnp.float32),
    scratch_types=[
        pltpu.VMEM((NCH, CH), jnp.int32),
        pltpu.VMEM((CH, H), jnp.float32),
        pltpu.SemaphoreType.DMA,
    ],
)
def _sc_gather(table, src2, out, idx_v, rows_v, sem):
    cid = lax.axis_index("c")
    sid = lax.axis_index("s")
    wid = sid * NC + cid
    pltpu.sync_copy(src2.at[wid], idx_v)

    def body(j, carry):
        base = wid * EPW + j * CH
        pltpu.async_copy(table.at[idx_v.at[j]], rows_v, sem).wait()
        pltpu.sync_copy(rows_v, out.at[pl.ds(base, CH), :])
        return carry

    lax.fori_loop(0, NCH, body, 0)


@functools.partial(
    pl.kernel,
    mesh=_mesh,
    out_type=jax.ShapeDtypeStruct((NC, NPAD, H), jnp.float32),
    scratch_types=[
        pltpu.VMEM((NCH, CH), jnp.int32),
        pltpu.VMEM((CH, H), jnp.float32),
        pltpu.VMEM_SHARED((NPAD, H), jnp.float32),
    ],
)
def _sc_scatter(msg, dst2, zeros128, out, idx_v, rows_v, acc):
    cid = lax.axis_index("c")
    sid = lax.axis_index("s")
    wid = sid * NC + cid
    pltpu.sync_copy(zeros128, rows_v)
    for k in range(RPT // CH):
        pltpu.sync_copy(rows_v, acc.at[pl.ds(sid * RPT + k * CH, CH), :])
    pltpu.sync_copy(dst2.at[wid], idx_v)
    plsc.subcore_barrier()

    def body(j, carry):
        base = wid * EPW + j * CH
        pltpu.sync_copy(msg.at[pl.ds(base, CH), :], rows_v)
        pltpu.sync_copy(rows_v, acc.at[idx_v.at[j]], add=True)
        return carry

    lax.fori_loop(0, NCH, body, 0)
    plsc.subcore_barrier()
    rows_o = pl.ds(sid * RPT, RPT)
    pltpu.sync_copy(acc.at[rows_o, :], out.at[cid, rows_o, :])


def _dot(a, b):
    return jnp.dot(a, b, preferred_element_type=jnp.float32)


def _full(shape):
    return pl.BlockSpec(shape, lambda i: tuple(0 for _ in shape))


def _edge_enc_body(ea, w1, b1, w2, b2, w3, b3, out):
    h = jnp.maximum(_dot(ea[...], w1[...]) + b1[...], 0.0)
    h = jnp.maximum(_dot(h, w2[...]) + b2[...], 0.0)
    out[...] = _dot(h, w3[...]) + b3[...]


def _tc_edge_enc(ea8, w1, b1, w2, b2, w3, b3):
    return pl.pallas_call(
        _edge_enc_body,
        grid=(E_PAD // TE,),
        in_specs=[pl.BlockSpec((TE, 8), lambda i: (i, 0)),
                  _full((8, H)), _full((1, H)),
                  _full((H, H)), _full((1, H)),
                  _full((H, H)), _full((1, H))],
        out_specs=pl.BlockSpec((TE, H), lambda i: (i, 0)),
        out_shape=jax.ShapeDtypeStruct((E_PAD, H), jnp.float32),
    )(ea8, w1, b1, w2, b2, w3, b3)


def _node_enc_body(x, c0, c1, w1, b1, w2, b2, w3, b3, wa, ba, emb, den, proj):
    col = lax.broadcasted_iota(jnp.int32, (TN, NF), 1)
    xm = jnp.where(col < 2, 0.0, x[...])
    h = jnp.maximum(_dot(xm, w1[...]) + b1[...], 0.0)
    h = jnp.maximum(_dot(h, w2[...]) + b2[...], 0.0)
    e = _dot(h, w3[...]) + b3[...]
    emb[...] = e
    cnt = c0[...] + c1[...]
    den[...] = jnp.broadcast_to(jnp.maximum(cnt[:, 0:1], 1.0), (TN, H))
    proj[...] = _dot(e, wa[...]) + ba[...]


def _tc_node_enc(x_p, c0, c1, w1, b1, w2, b2, w3, b3, wa, ba):
    return pl.pallas_call(
        _node_enc_body,
        grid=(NPAD // TN,),
        in_specs=[pl.BlockSpec((TN, NF), lambda i: (i, 0)),
                  pl.BlockSpec((TN, H), lambda i: (i, 0)),
                  pl.BlockSpec((TN, H), lambda i: (i, 0)),
                  _full((NF, H)), _full((1, H)),
                  _full((H, H)), _full((1, H)),
                  _full((H, H)), _full((1, H)),
                  _full((H, H)), _full((1, H))],
        out_specs=[pl.BlockSpec((TN, H), lambda i: (i, 0)),
                   pl.BlockSpec((TN, H), lambda i: (i, 0)),
                   pl.BlockSpec((TN, H), lambda i: (i, 0))],
        out_shape=[jax.ShapeDtypeStruct((NPAD, H), jnp.float32),
                   jax.ShapeDtypeStruct((NPAD, H), jnp.float32),
                   jax.ShapeDtypeStruct((NPAD, H), jnp.float32)],
    )(x_p, c0, c1, w1, b1, w2, b2, w3, b3, wa, ba)


def _msg_body(xjp, ee, wb, w2, b2, w3, b3, out):
    h1 = jnp.maximum(xjp[...] + _dot(ee[...], wb[...]), 0.0)
    h2 = jnp.maximum(_dot(h1, w2[...]) + b2[...], 0.0)
    out[...] = _dot(h2, w3[...]) + b3[...]


def _tc_msg(xjp, ee, wb, w2, b2, w3, b3):
    return pl.pallas_call(
        _msg_body,
        grid=(E_PAD // TE,),
        in_specs=[pl.BlockSpec((TE, H), lambda i: (i, 0)),
                  pl.BlockSpec((TE, H), lambda i: (i, 0)),
                  _full((H, H)), _full((H, H)), _full((1, H)),
                  _full((H, H)), _full((1, H))],
        out_specs=pl.BlockSpec((TE, H), lambda i: (i, 0)),
        out_shape=jax.ShapeDtypeStruct((E_PAD, H), jnp.float32),
    )(xjp, ee, wb, w2, b2, w3, b3)


def _update_body(emb, p0, p1, den, wa, ba, nemb, proj):
    e = emb[...] + (p0[...] + p1[...]) / den[...]
    nemb[...] = e
    proj[...] = _dot(e, wa[...]) + ba[...]


def _tc_update(emb, p0, p1, den, wa, ba):
    return pl.pallas_call(
        _update_body,
        grid=(NPAD // TN,),
        in_specs=[pl.BlockSpec((TN, H), lambda i: (i, 0)),
                  pl.BlockSpec((TN, H), lambda i: (i, 0)),
                  pl.BlockSpec((TN, H), lambda i: (i, 0)),
                  pl.BlockSpec((TN, H), lambda i: (i, 0)),
                  _full((H, H)), _full((1, H))],
        out_specs=[pl.BlockSpec((TN, H), lambda i: (i, 0)),
                   pl.BlockSpec((TN, H), lambda i: (i, 0))],
        out_shape=[jax.ShapeDtypeStruct((NPAD, H), jnp.float32),
                   jax.ShapeDtypeStruct((NPAD, H), jnp.float32)],
    )(emb, p0, p1, den, wa, ba)


def _final_body(emb, p0, p1, den, w1, b1, w2, b2, w3, b3, out):
    e = emb[...] + (p0[...] + p1[...]) / den[...]
    h = jnp.maximum(_dot(e, w1[...]) + b1[...], 0.0)
    h = jnp.maximum(_dot(h, w2[...]) + b2[...], 0.0)
    out[...] = _dot(h, w3[...]) + b3[...]


def _tc_final(emb, p0, p1, den, w1, b1, w2, b2, w3, b3):
    return pl.pallas_call(
        _final_body,
        grid=(NPAD // TN,),
        in_specs=[pl.BlockSpec((TN, H), lambda i: (i, 0)),
                  pl.BlockSpec((TN, H), lambda i: (i, 0)),
                  pl.BlockSpec((TN, H), lambda i: (i, 0)),
                  pl.BlockSpec((TN, H), lambda i: (i, 0)),
                  _full((H, H)), _full((1, H)),
                  _full((H, H)), _full((1, H)),
                  _full((H, H)), _full((1, H))],
        out_specs=pl.BlockSpec((TN, H), lambda i: (i, 0)),
        out_shape=jax.ShapeDtypeStruct((NPAD, H), jnp.float32),
    )(emb, p0, p1, den, w1, b1, w2, b2, w3, b3)


def _row(b):
    return b.reshape(1, H)


def kernel(x, edge_index, edge_attr, params):
    src = edge_index[0].astype(jnp.int32)
    dst = edge_index[1].astype(jnp.int32)
    pad_idx = (N + jnp.arange(E_PAD - E, dtype=jnp.int32) % (NPAD - N))
    src2 = jnp.concatenate([src, pad_idx]).reshape(NW, NCH, CH)
    dst2 = jnp.concatenate([dst, pad_idx]).reshape(NW, NCH, CH)
    x_p = jnp.pad(x, ((0, NPAD - N), (0, 0)))
    ea8 = jnp.pad(edge_attr, ((0, E_PAD - E), (0, 8 - EF)))

    nps = params["node"]
    eps = params["edge"]
    dec = params["dec"]
    procs = params["proc"]

    ew1 = jnp.pad(eps[0][0], ((0, 8 - EF), (0, 0)))
    dw3 = jnp.pad(dec[2][0], ((0, 0), (0, H - OUT)))
    db3 = jnp.pad(dec[2][1], ((0, H - OUT),))

    was = [p[0][0][:H] for p in procs]
    wbs = [p[0][0][H:] for p in procs]

    ones_rows = jnp.zeros((CH, H), jnp.float32).at[:, 0].set(1.0)
    zeros128 = jnp.zeros((CH, H), jnp.float32)

    cpart = _sc_counts(dst2, ones_rows, zeros128)
    c0, c1 = cpart[0], cpart[1]
    edge_emb = _tc_edge_enc(ea8, ew1, _row(eps[0][1]),
                            eps[1][0], _row(eps[1][1]),
                            eps[2][0], _row(eps[2][1]))
    emb, den, proj = _tc_node_enc(
        x_p, c0, c1,
        nps[0][0], _row(nps[0][1]),
        nps[1][0], _row(nps[1][1]),
        nps[2][0], _row(nps[2][1]),
        was[0], _row(procs[0][0][1]))

    out = None
    for s in range(STEPS):
        ps = procs[s]
        xjp = _sc_gather(proj, src2)
        msg = _tc_msg(xjp, edge_emb, wbs[s],
                      ps[1][0], _row(ps[1][1]),
                      ps[2][0], _row(ps[2][1]))
        part = _sc_scatter(msg, dst2, zeros128)
        p0, p1 = part[0], part[1]
        if s + 1 < STEPS:
            emb, proj = _tc_update(emb, p0, p1, den,
                                   was[s + 1], _row(procs[s + 1][0][1]))
        else:
            out = _tc_final(emb, p0, p1, den,
                            dec[0][0], _row(dec[0][1]),
                            dec[1][0], _row(dec[1][1]),
                            dw3, _row(db3))
    return out[:N, :OUT]


# speedup vs baseline: 1.7129x; 1.1045x over previous
import functools

import jax
import jax.numpy as jnp
from jax import lax
from jax.experimental import pallas as pl
from jax.experimental.pallas import tpu as pltpu
from jax.experimental.pallas import tpu_sc as plsc

N = 10000
NPAD = 10240
E = 320000
NF = 16
EF = 3
H = 128
OUT = 2
STEPS = 10

NC = 2
NS = 16
NW = NC * NS
E_PAD = 327680
EPW = E_PAD // NW
CH = 128
NCH = EPW // CH
RPT = NPAD // NS

TN = 1024
TE = 2560

_mesh = plsc.VectorSubcoreMesh(core_axis_name="c", subcore_axis_name="s")


@functools.partial(
    pl.kernel,
    mesh=_mesh,
    out_type=jax.ShapeDtypeStruct((NC, NPAD, H), jnp.float32),
    scratch_types=[
        pltpu.VMEM((NCH, CH), jnp.int32),
        pltpu.VMEM((CH, H), jnp.float32),
        pltpu.VMEM_SHARED((NPAD, H), jnp.float32),
    ],
)
def _sc_counts(dst2, ones_rows, zeros128, out, idx_v, ones_v, acc):
    cid = lax.axis_index("c")
    sid = lax.axis_index("s")
    wid = sid * NC + cid
    pltpu.sync_copy(zeros128, ones_v)
    for k in range(RPT // CH):
        pltpu.sync_copy(ones_v, acc.at[pl.ds(sid * RPT + k * CH, CH), :])
    pltpu.sync_copy(ones_rows, ones_v)
    pltpu.sync_copy(dst2.at[wid], idx_v)
    plsc.subcore_barrier()

    def body(j, carry):
        pltpu.sync_copy(ones_v, acc.at[idx_v.at[j]], add=True)
        return carry

    lax.fori_loop(0, NCH, body, 0)
    plsc.subcore_barrier()
    rows = pl.ds(sid * RPT, RPT)
    pltpu.sync_copy(acc.at[rows, :], out.at[cid, rows, :])


@functools.partial(
    pl.kernel,
    mesh=_mesh,
    out_type=jax.ShapeDtypeStruct((E_PAD, H), jnp.float32),
    scratch_types=[
        pltpu.VMEM((NCH, CH), jnp.int32),
        pltpu.VMEM((CH, H), jnp.float32),
        pltpu.SemaphoreType.DMA,
    ],
)
def _sc_gather(table, src2, out, idx_v, rows_v, sem):
    cid = lax.axis_index("c")
    sid = lax.axis_index("s")
    wid = sid * NC + cid
    pltpu.sync_copy(src2.at[wid], idx_v)

    def body(j, carry):
        base = wid * EPW + j * CH
        pltpu.async_copy(table.at[idx_v.at[j]], rows_v, sem).wait()
        pltpu.sync_copy(rows_v, out.at[pl.ds(base, CH), :])
        return carry

    lax.fori_loop(0, NCH, body, 0)


@functools.partial(
    pl.kernel,
    mesh=_mesh,
    out_type=jax.ShapeDtypeStruct((NC, NPAD, H), jnp.float32),
    scratch_types=[
        pltpu.VMEM((NCH, CH), jnp.int32),
        pltpu.VMEM((CH, H), jnp.float32),
        pltpu.VMEM_SHARED((NPAD, H), jnp.float32),
    ],
)
def _sc_scatter(msg, dst2, zeros128, out, idx_v, rows_v, acc):
    cid = lax.axis_index("c")
    sid = lax.axis_index("s")
    wid = sid * NC + cid
    pltpu.sync_copy(zeros128, rows_v)
    for k in range(RPT // CH):
        pltpu.sync_copy(rows_v, acc.at[pl.ds(sid * RPT + k * CH, CH), :])
    pltpu.sync_copy(dst2.at[wid], idx_v)
    plsc.subcore_barrier()

    def body(j, carry):
        base = wid * EPW + j * CH
        pltpu.sync_copy(msg.at[pl.ds(base, CH), :], rows_v)
        pltpu.sync_copy(rows_v, acc.at[idx_v.at[j]], add=True)
        return carry

    lax.fori_loop(0, NCH, body, 0)
    plsc.subcore_barrier()
    rows_o = pl.ds(sid * RPT, RPT)
    pltpu.sync_copy(acc.at[rows_o, :], out.at[cid, rows_o, :])


def _dot(a, b):
    return jnp.dot(a, b, preferred_element_type=jnp.float32)


def _full(shape):
    return pl.BlockSpec(shape, lambda i: tuple(0 for _ in shape))


def _edge_enc_body(ea, w1, b1, w2, b2, w3, b3, out):
    h = jnp.maximum(_dot(ea[...], w1[...]) + b1[...], 0.0)
    h = jnp.maximum(_dot(h, w2[...]) + b2[...], 0.0)
    out[...] = _dot(h, w3[...]) + b3[...]


def _tc_edge_enc(ea8, w1, b1, w2, b2, w3, b3):
    return pl.pallas_call(
        _edge_enc_body,
        grid=(E_PAD // TE,),
        in_specs=[pl.BlockSpec((TE, 8), lambda i: (i, 0)),
                  _full((8, H)), _full((1, H)),
                  _full((H, H)), _full((1, H)),
                  _full((H, H)), _full((1, H))],
        out_specs=pl.BlockSpec((TE, H), lambda i: (i, 0)),
        out_shape=jax.ShapeDtypeStruct((E_PAD, H), jnp.float32),
    )(ea8, w1, b1, w2, b2, w3, b3)


def _node_enc_body(x, c0, c1, w1, b1, w2, b2, w3, b3, wa, ba, emb, den, proj):
    col = lax.broadcasted_iota(jnp.int32, (TN, NF), 1)
    xm = jnp.where(col < 2, 0.0, x[...])
    h = jnp.maximum(_dot(xm, w1[...]) + b1[...], 0.0)
    h = jnp.maximum(_dot(h, w2[...]) + b2[...], 0.0)
    e = _dot(h, w3[...]) + b3[...]
    emb[...] = e
    cnt = c0[...] + c1[...]
    den[...] = jnp.broadcast_to(jnp.maximum(cnt[:, 0:1], 1.0), (TN, H))
    proj[...] = _dot(e, wa[...]) + ba[...]


def _tc_node_enc(x_p, c0, c1, w1, b1, w2, b2, w3, b3, wa, ba):
    return pl.pallas_call(
        _node_enc_body,
        grid=(NPAD // TN,),
        in_specs=[pl.BlockSpec((TN, NF), lambda i: (i, 0)),
                  pl.BlockSpec((TN, H), lambda i: (i, 0)),
                  pl.BlockSpec((TN, H), lambda i: (i, 0)),
                  _full((NF, H)), _full((1, H)),
                  _full((H, H)), _full((1, H)),
                  _full((H, H)), _full((1, H)),
                  _full((H, H)), _full((1, H))],
        out_specs=[pl.BlockSpec((TN, H), lambda i: (i, 0)),
                   pl.BlockSpec((TN, H), lambda i: (i, 0)),
                   pl.BlockSpec((TN, H), lambda i: (i, 0))],
        out_shape=[jax.ShapeDtypeStruct((NPAD, H), jnp.float32),
                   jax.ShapeDtypeStruct((NPAD, H), jnp.float32),
                   jax.ShapeDtypeStruct((NPAD, H), jnp.float32)],
    )(x_p, c0, c1, w1, b1, w2, b2, w3, b3, wa, ba)


def _msg_body(xjp, ee, wb, w2, b2, w3, b3, out):
    h1 = jnp.maximum(xjp[...] + _dot(ee[...], wb[...]), 0.0)
    h2 = jnp.maximum(_dot(h1, w2[...]) + b2[...], 0.0)
    out[...] = _dot(h2, w3[...]) + b3[...]


def _tc_msg(xjp, ee, wb, w2, b2, w3, b3):
    return pl.pallas_call(
        _msg_body,
        grid=(E_PAD // TE,),
        in_specs=[pl.BlockSpec((TE, H), lambda i: (i, 0)),
                  pl.BlockSpec((TE, H), lambda i: (i, 0)),
                  _full((H, H)), _full((H, H)), _full((1, H)),
                  _full((H, H)), _full((1, H))],
        out_specs=pl.BlockSpec((TE, H), lambda i: (i, 0)),
        out_shape=jax.ShapeDtypeStruct((E_PAD, H), jnp.float32),
    )(xjp, ee, wb, w2, b2, w3, b3)


def _update_body(emb, p0, p1, den, wa, ba, nemb, proj):
    e = emb[...] + (p0[...] + p1[...]) / den[...]
    nemb[...] = e
    proj[...] = _dot(e, wa[...]) + ba[...]


def _tc_update(emb, p0, p1, den, wa, ba):
    return pl.pallas_call(
        _update_body,
        grid=(NPAD // TN,),
        in_specs=[pl.BlockSpec((TN, H), lambda i: (i, 0)),
                  pl.BlockSpec((TN, H), lambda i: (i, 0)),
                  pl.BlockSpec((TN, H), lambda i: (i, 0)),
                  pl.BlockSpec((TN, H), lambda i: (i, 0)),
                  _full((H, H)), _full((1, H))],
        out_specs=[pl.BlockSpec((TN, H), lambda i: (i, 0)),
                   pl.BlockSpec((TN, H), lambda i: (i, 0))],
        out_shape=[jax.ShapeDtypeStruct((NPAD, H), jnp.float32),
                   jax.ShapeDtypeStruct((NPAD, H), jnp.float32)],
    )(emb, p0, p1, den, wa, ba)


def _final_body(emb, p0, p1, den, w1, b1, w2, b2, w3, b3, out):
    e = emb[...] + (p0[...] + p1[...]) / den[...]
    h = jnp.maximum(_dot(e, w1[...]) + b1[...], 0.0)
    h = jnp.maximum(_dot(h, w2[...]) + b2[...], 0.0)
    out[...] = _dot(h, w3[...]) + b3[...]


def _tc_final(emb, p0, p1, den, w1, b1, w2, b2, w3, b3):
    return pl.pallas_call(
        _final_body,
        grid=(NPAD // TN,),
        in_specs=[pl.BlockSpec((TN, H), lambda i: (i, 0)),
                  pl.BlockSpec((TN, H), lambda i: (i, 0)),
                  pl.BlockSpec((TN, H), lambda i: (i, 0)),
                  pl.BlockSpec((TN, H), lambda i: (i, 0)),
                  _full((H, H)), _full((1, H)),
                  _full((H, H)), _full((1, H)),
                  _full((H, H)), _full((1, H))],
        out_specs=pl.BlockSpec((TN, H), lambda i: (i, 0)),
        out_shape=jax.ShapeDtypeStruct((NPAD, H), jnp.float32),
    )(emb, p0, p1, den, w1, b1, w2, b2, w3, b3)


def _row(b):
    return b.reshape(1, H)


def kernel(x, edge_index, edge_attr, params):
    src = edge_index[0].astype(jnp.int32)
    dst = edge_index[1].astype(jnp.int32)
    pad_idx = (N + jnp.arange(E_PAD - E, dtype=jnp.int32) % (NPAD - N))
    src2 = jnp.concatenate([src, pad_idx]).reshape(NW, NCH, CH)
    dst2 = jnp.concatenate([dst, pad_idx]).reshape(NW, NCH, CH)
    x_p = jnp.pad(x, ((0, NPAD - N), (0, 0)))
    ea8 = jnp.pad(edge_attr, ((0, E_PAD - E), (0, 8 - EF)))

    nps = params["node"]
    eps = params["edge"]
    dec = params["dec"]
    procs = params["proc"]

    ew1 = jnp.pad(eps[0][0], ((0, 8 - EF), (0, 0)))
    dw3 = jnp.pad(dec[2][0], ((0, 0), (0, H - OUT)))
    db3 = jnp.pad(dec[2][1], ((0, H - OUT),))

    was = [p[0][0][:H] for p in procs]
    wbs = [p[0][0][H:] for p in procs]

    ones_rows = jnp.zeros((CH, H), jnp.float32).at[:, 0].set(1.0)
    zeros128 = jnp.zeros((CH, H), jnp.float32)

    cpart = _sc_counts(dst2, ones_rows, zeros128)
    c0, c1 = cpart[0], cpart[1]
    edge_emb = _tc_edge_enc(ea8, ew1, _row(eps[0][1]),
                            eps[1][0], _row(eps[1][1]),
                            eps[2][0], _row(eps[2][1]))
    emb, den, proj = _tc_node_enc(
        x_p, c0, c1,
        nps[0][0], _row(nps[0][1]),
        nps[1][0], _row(nps[1][1]),
        nps[2][0], _row(nps[2][1]),
        was[0], _row(procs[0][0][1]))

    out = None
    for s in range(STEPS):
        ps = procs[s]
        xjp = _sc_gather(proj, src2)
        msg = _tc_msg(xjp, edge_emb, wbs[s],
                      ps[1][0], _row(ps[1][1]),
                      ps[2][0], _row(ps[2][1]))
        part = _sc_scatter(msg, dst2, zeros128)
        p0, p1 = part[0], part[1]
        if s + 1 < STEPS:
            emb, proj = _tc_update(emb, p0, p1, den,
                                   was[s + 1], _row(procs[s + 1][0][1]))
        else:
            out = _tc_final(emb, p0, p1, den,
                            dec[0][0], _row(dec[0][1]),
                            dec[1][0], _row(dec[1][1]),
                            dw3, _row(db3))
    return out[:N, :OUT]


# speedup vs baseline: 2.0302x; 1.1852x over previous
import functools

import jax
import jax.numpy as jnp
from jax import lax
from jax.experimental import pallas as pl
from jax.experimental.pallas import tpu as pltpu
from jax.experimental.pallas import tpu_sc as plsc

N = 10000
NPAD = 10240
E = 320000
NF = 16
EF = 3
H = 128
OUT = 2
STEPS = 10

NC = 2
NS = 16
NW = NC * NS
E_PAD = 327680
EPW = E_PAD // NW
CH = 128
NCH = EPW // CH
RPT = NPAD // NS

TN = 1024
TE = 2560

_mesh = plsc.VectorSubcoreMesh(core_axis_name="c", subcore_axis_name="s")


@functools.partial(
    pl.kernel,
    mesh=_mesh,
    out_type=jax.ShapeDtypeStruct((NC, NPAD, H), jnp.float32),
    scratch_types=[
        pltpu.VMEM((NCH, CH), jnp.int32),
        pltpu.VMEM((CH, H), jnp.float32),
        pltpu.VMEM_SHARED((NPAD, H), jnp.float32),
    ],
)
def _sc_counts(dst2, ones_rows, zeros128, out, idx_v, ones_v, acc):
    cid = lax.axis_index("c")
    sid = lax.axis_index("s")
    wid = sid * NC + cid
    pltpu.sync_copy(zeros128, ones_v)
    for k in range(RPT // CH):
        pltpu.sync_copy(ones_v, acc.at[pl.ds(sid * RPT + k * CH, CH), :])
    pltpu.sync_copy(ones_rows, ones_v)
    pltpu.sync_copy(dst2.at[wid], idx_v)
    plsc.subcore_barrier()

    def body(j, carry):
        pltpu.sync_copy(ones_v, acc.at[idx_v.at[j]], add=True)
        return carry

    lax.fori_loop(0, NCH, body, 0)
    plsc.subcore_barrier()
    rows = pl.ds(sid * RPT, RPT)
    pltpu.sync_copy(acc.at[rows, :], out.at[cid, rows, :])


def _make_sc_gather(ne):
    epw = ne // NW
    nch = epw // CH

    @functools.partial(
        pl.kernel,
        mesh=_mesh,
        out_type=jax.ShapeDtypeStruct((ne, H), jnp.float32),
        scratch_types=[
            pltpu.VMEM((nch, CH), jnp.int32),
            pltpu.VMEM((CH, H), jnp.float32),
            pltpu.SemaphoreType.DMA,
        ],
    )
    def _sc_gather(table, src2, out, idx_v, rows_v, sem):
        cid = lax.axis_index("c")
        sid = lax.axis_index("s")
        wid = sid * NC + cid
        pltpu.sync_copy(src2.at[wid], idx_v)

        def body(j, carry):
            base = wid * epw + j * CH
            pltpu.async_copy(table.at[idx_v.at[j]], rows_v, sem).wait()
            pltpu.sync_copy(rows_v, out.at[pl.ds(base, CH), :])
            return carry

        lax.fori_loop(0, nch, body, 0)

    return _sc_gather


_sc_gather_half = _make_sc_gather(E_PAD // 2)


def _make_sc_scatter(ne):
    epw = ne // NW
    nch = epw // CH

    @functools.partial(
        pl.kernel,
        mesh=_mesh,
        out_type=jax.ShapeDtypeStruct((NC, NPAD, H), jnp.float32),
        scratch_types=[
            pltpu.VMEM((nch, CH), jnp.int32),
            pltpu.VMEM((CH, H), jnp.float32),
            pltpu.VMEM_SHARED((NPAD, H), jnp.float32),
        ],
    )
    def _sc_scatter(msg, dst2, zeros128, out, idx_v, rows_v, acc):
        cid = lax.axis_index("c")
        sid = lax.axis_index("s")
        wid = sid * NC + cid
        pltpu.sync_copy(zeros128, rows_v)
        for k in range(RPT // CH):
            pltpu.sync_copy(rows_v, acc.at[pl.ds(sid * RPT + k * CH, CH), :])
        pltpu.sync_copy(dst2.at[wid], idx_v)
        plsc.subcore_barrier()

        def body(j, carry):
            base = wid * epw + j * CH
            pltpu.sync_copy(msg.at[pl.ds(base, CH), :], rows_v)
            pltpu.sync_copy(rows_v, acc.at[idx_v.at[j]], add=True)
            return carry

        lax.fori_loop(0, nch, body, 0)
        plsc.subcore_barrier()
        rows_o = pl.ds(sid * RPT, RPT)
        pltpu.sync_copy(acc.at[rows_o, :], out.at[cid, rows_o, :])

    return _sc_scatter


_sc_scatter_half = _make_sc_scatter(E_PAD // 2)


def _dot(a, b):
    return jnp.dot(a, b, preferred_element_type=jnp.float32)


def _full(shape):
    return pl.BlockSpec(shape, lambda i: tuple(0 for _ in shape))


def _edge_enc_body(ea, w1, b1, w2, b2, w3, b3, out):
    h = jnp.maximum(_dot(ea[...], w1[...]) + b1[...], 0.0)
    h = jnp.maximum(_dot(h, w2[...]) + b2[...], 0.0)
    out[...] = _dot(h, w3[...]) + b3[...]


def _tc_edge_enc(ea8, w1, b1, w2, b2, w3, b3):
    ne = ea8.shape[0]
    return pl.pallas_call(
        _edge_enc_body,
        grid=(ne // TE,),
        in_specs=[pl.BlockSpec((TE, 8), lambda i: (i, 0)),
                  _full((8, H)), _full((1, H)),
                  _full((H, H)), _full((1, H)),
                  _full((H, H)), _full((1, H))],
        out_specs=pl.BlockSpec((TE, H), lambda i: (i, 0)),
        out_shape=jax.ShapeDtypeStruct((ne, H), jnp.float32),
    )(ea8, w1, b1, w2, b2, w3, b3)


def _node_enc_body(x, c0, c1, w1, b1, w2, b2, w3, b3, wa, ba, emb, den, proj):
    col = lax.broadcasted_iota(jnp.int32, (TN, NF), 1)
    xm = jnp.where(col < 2, 0.0, x[...])
    h = jnp.maximum(_dot(xm, w1[...]) + b1[...], 0.0)
    h = jnp.maximum(_dot(h, w2[...]) + b2[...], 0.0)
    e = _dot(h, w3[...]) + b3[...]
    emb[...] = e
    cnt = c0[...] + c1[...]
    den[...] = jnp.broadcast_to(jnp.maximum(cnt[:, 0:1], 1.0), (TN, H))
    proj[...] = _dot(e, wa[...]) + ba[...]


def _tc_node_enc(x_p, c0, c1, w1, b1, w2, b2, w3, b3, wa, ba):
    return pl.pallas_call(
        _node_enc_body,
        grid=(NPAD // TN,),
        in_specs=[pl.BlockSpec((TN, NF), lambda i: (i, 0)),
                  pl.BlockSpec((TN, H), lambda i: (i, 0)),
                  pl.BlockSpec((TN, H), lambda i: (i, 0)),
                  _full((NF, H)), _full((1, H)),
                  _full((H, H)), _full((1, H)),
                  _full((H, H)), _full((1, H)),
                  _full((H, H)), _full((1, H))],
        out_specs=[pl.BlockSpec((TN, H), lambda i: (i, 0)),
                   pl.BlockSpec((TN, H), lambda i: (i, 0)),
                   pl.BlockSpec((TN, H), lambda i: (i, 0))],
        out_shape=[jax.ShapeDtypeStruct((NPAD, H), jnp.float32),
                   jax.ShapeDtypeStruct((NPAD, H), jnp.float32),
                   jax.ShapeDtypeStruct((NPAD, H), jnp.float32)],
    )(x_p, c0, c1, w1, b1, w2, b2, w3, b3, wa, ba)


def _msg_body(xjp, ee, wb, w2, b2, w3, b3, out):
    h1 = jnp.maximum(xjp[...] + _dot(ee[...], wb[...]), 0.0)
    h2 = jnp.maximum(_dot(h1, w2[...]) + b2[...], 0.0)
    out[...] = _dot(h2, w3[...]) + b3[...]


def _tc_msg(xjp, ee, wb, w2, b2, w3, b3):
    ne = xjp.shape[0]
    return pl.pallas_call(
        _msg_body,
        grid=(ne // TE,),
        in_specs=[pl.BlockSpec((TE, H), lambda i: (i, 0)),
                  pl.BlockSpec((TE, H), lambda i: (i, 0)),
                  _full((H, H)), _full((H, H)), _full((1, H)),
                  _full((H, H)), _full((1, H))],
        out_specs=pl.BlockSpec((TE, H), lambda i: (i, 0)),
        out_shape=jax.ShapeDtypeStruct((ne, H), jnp.float32),
    )(xjp, ee, wb, w2, b2, w3, b3)


def _update_body(emb, p0, p1, p2, p3, den, wa, ba, nemb, proj):
    e = emb[...] + (p0[...] + p1[...] + p2[...] + p3[...]) / den[...]
    nemb[...] = e
    proj[...] = _dot(e, wa[...]) + ba[...]


def _tc_update(emb, p0, p1, p2, p3, den, wa, ba):
    return pl.pallas_call(
        _update_body,
        grid=(NPAD // TN,),
        in_specs=[pl.BlockSpec((TN, H), lambda i: (i, 0))] * 6 +
                 [_full((H, H)), _full((1, H))],
        out_specs=[pl.BlockSpec((TN, H), lambda i: (i, 0)),
                   pl.BlockSpec((TN, H), lambda i: (i, 0))],
        out_shape=[jax.ShapeDtypeStruct((NPAD, H), jnp.float32),
                   jax.ShapeDtypeStruct((NPAD, H), jnp.float32)],
    )(emb, p0, p1, p2, p3, den, wa, ba)


def _final_body(emb, p0, p1, p2, p3, den, w1, b1, w2, b2, w3, b3, out):
    e = emb[...] + (p0[...] + p1[...] + p2[...] + p3[...]) / den[...]
    h = jnp.maximum(_dot(e, w1[...]) + b1[...], 0.0)
    h = jnp.maximum(_dot(h, w2[...]) + b2[...], 0.0)
    out[...] = _dot(h, w3[...]) + b3[...]


def _tc_final(emb, p0, p1, p2, p3, den, w1, b1, w2, b2, w3, b3):
    return pl.pallas_call(
        _final_body,
        grid=(NPAD // TN,),
        in_specs=[pl.BlockSpec((TN, H), lambda i: (i, 0))] * 6 +
                 [_full((H, H)), _full((1, H)),
                  _full((H, H)), _full((1, H)),
                  _full((H, H)), _full((1, H))],
        out_specs=pl.BlockSpec((TN, H), lambda i: (i, 0)),
        out_shape=jax.ShapeDtypeStruct((NPAD, H), jnp.float32),
    )(emb, p0, p1, p2, p3, den, w1, b1, w2, b2, w3, b3)


def _row(b):
    return b.reshape(1, H)


def kernel(x, edge_index, edge_attr, params):
    src = edge_index[0].astype(jnp.int32)
    dst = edge_index[1].astype(jnp.int32)
    eh = E_PAD // 2
    nch_h = eh // NW // CH
    pad_idx = (N + jnp.arange(E_PAD - E, dtype=jnp.int32) % (NPAD - N))
    src_p = jnp.concatenate([src, pad_idx])
    dst_p = jnp.concatenate([dst, pad_idx])
    dst2 = dst_p.reshape(NW, NCH, CH)
    srcA = src_p[:eh].reshape(NW, nch_h, CH)
    srcB = src_p[eh:].reshape(NW, nch_h, CH)
    dstA = dst_p[:eh].reshape(NW, nch_h, CH)
    dstB = dst_p[eh:].reshape(NW, nch_h, CH)
    x_p = jnp.pad(x, ((0, NPAD - N), (0, 0)))
    ea8 = jnp.pad(edge_attr, ((0, E_PAD - E), (0, 8 - EF)))

    nps = params["node"]
    eps = params["edge"]
    dec = params["dec"]
    procs = params["proc"]

    ew1 = jnp.pad(eps[0][0], ((0, 8 - EF), (0, 0)))
    dw3 = jnp.pad(dec[2][0], ((0, 0), (0, H - OUT)))
    db3 = jnp.pad(dec[2][1], ((0, H - OUT),))

    was = [p[0][0][:H] for p in procs]
    wbs = [p[0][0][H:] for p in procs]

    ones_rows = jnp.zeros((CH, H), jnp.float32).at[:, 0].set(1.0)
    zeros128 = jnp.zeros((CH, H), jnp.float32)

    cpart = _sc_counts(dst2, ones_rows, zeros128)
    c0, c1 = cpart[0], cpart[1]
    eeA = _tc_edge_enc(ea8[:eh], ew1, _row(eps[0][1]),
                       eps[1][0], _row(eps[1][1]),
                       eps[2][0], _row(eps[2][1]))
    eeB = _tc_edge_enc(ea8[eh:], ew1, _row(eps[0][1]),
                       eps[1][0], _row(eps[1][1]),
                       eps[2][0], _row(eps[2][1]))
    emb, den, proj = _tc_node_enc(
        x_p, c0, c1,
        nps[0][0], _row(nps[0][1]),
        nps[1][0], _row(nps[1][1]),
        nps[2][0], _row(nps[2][1]),
        was[0], _row(procs[0][0][1]))

    out = None
    for s in range(STEPS):
        ps = procs[s]
        w2, b2 = ps[1][0], _row(ps[1][1])
        w3, b3 = ps[2][0], _row(ps[2][1])
        xjpA = _sc_gather_half(proj, srcA)
        xjpB = _sc_gather_half(proj, srcB)
        msgA = _tc_msg(xjpA, eeA, wbs[s], w2, b2, w3, b3)
        partA = _sc_scatter_half(msgA, dstA, zeros128)
        msgB = _tc_msg(xjpB, eeB, wbs[s], w2, b2, w3, b3)
        partB = _sc_scatter_half(msgB, dstB, zeros128)
        p0, p1, p2, p3 = partA[0], partA[1], partB[0], partB[1]
        if s + 1 < STEPS:
            emb, proj = _tc_update(emb, p0, p1, p2, p3, den,
                                   was[s + 1], _row(procs[s + 1][0][1]))
        else:
            out = _tc_final(emb, p0, p1, p2, p3, den,
                            dec[0][0], _row(dec[0][1]),
                            dec[1][0], _row(dec[1][1]),
                            dw3, _row(db3))
    return out[:N, :OUT]
